# baseline (speedup 1.0000x reference)
import jax
import jax.numpy as jnp
from jax.experimental import pallas as pl
from jax.experimental.pallas import tpu as pltpu

_LANE = 128
_SUBLANE = 8
_TARGET_BLOCK_BYTES = 8 * 2**20


def _soca4d_kernel(inv_hw, alpha_ref, x_ref, w1_ref, b1_ref, w2_ref, b2_ref,
                   o_ref):
    x = x_ref[...]

    t = jnp.sum(x.astype(jnp.float32), axis=2)
    pooled = jnp.sum(t, axis=-1) * inv_hw

    h = jax.lax.dot_general(
        pooled, w1_ref[...], (((1,), (1,)), ((), ())),
        preferred_element_type=jnp.float32) + b1_ref[...]
    a = alpha_ref[0]
    h = jnp.maximum(h, 0.0) + a * jnp.minimum(h, 0.0)
    z = jax.lax.dot_general(
        h, w2_ref[...], (((1,), (1,)), ((), ())),
        preferred_element_type=jnp.float32) + b2_ref[...]
    s = jax.nn.sigmoid(z).astype(o_ref.dtype)

    s4 = jax.lax.broadcast_in_dim(s, x.shape, (0, 1))
    o_ref[...] = x * s4


def kernel(x, w1, b1, w2, b2, alpha):
    B, C, H, W = x.shape
    hidden = w1.shape[0]
    dtype = x.dtype
    itemsize = jnp.dtype(dtype).itemsize

    W_pad = -(-W // _LANE) * _LANE
    H_pad = -(-H // _SUBLANE) * _SUBLANE
    row_bytes = C * H_pad * W_pad * itemsize

    bt = 1
    for cand in range(1, B + 1):
        if B % cand == 0 and cand * row_bytes <= _TARGET_BLOCK_BYTES:
            bt = cand
    nb = B // bt

    inv_hw = 1.0 / float(H * W)
    b1r = b1.astype(jnp.float32).reshape(1, hidden)
    b2r = b2.astype(jnp.float32).reshape(1, C)
    alpha_f = alpha.astype(jnp.float32).reshape(1)

    block_bytes = bt * row_bytes
    vmem = int(min(100 * 2**20, 4 * block_bytes + 6 * 2**20))

    import functools
    return pl.pallas_call(
        functools.partial(_soca4d_kernel, inv_hw),
        out_shape=jax.ShapeDtypeStruct((B, C, H, W), dtype),
        grid=(nb,),
        in_specs=[
            pl.BlockSpec(memory_space=pltpu.MemorySpace.SMEM),
            pl.BlockSpec((bt, C, H, W), lambda b: (b, 0, 0, 0)),
            pl.BlockSpec((hidden, C), lambda b: (0, 0)),
            pl.BlockSpec((1, hidden), lambda b: (0, 0)),
            pl.BlockSpec((C, hidden), lambda b: (0, 0)),
            pl.BlockSpec((1, C), lambda b: (0, 0)),
        ],
        out_specs=pl.BlockSpec((bt, C, H, W), lambda b: (b, 0, 0, 0)),
        compiler_params=pltpu.CompilerParams(
            dimension_semantics=("parallel",),
            vmem_limit_bytes=vmem),
    )(alpha_f, x, w1, b1r, w2, b2r)


# speedup vs baseline: 1.0047x; 1.0047x over previous
import jax
import jax.numpy as jnp
from jax.experimental import pallas as pl
from jax.experimental.pallas import tpu as pltpu

_LANE = 128
_SUBLANE = 8
_TARGET_BLOCK_BYTES = 8 * 2**20


def _soca4d_kernel(inv_hw, alpha_ref, x_ref, w1_ref, b1_ref, w2_ref, b2_ref,
                   o_ref):
    x = x_ref[...]

    t = jnp.sum(x.astype(jnp.float32), axis=2)
    pooled = jnp.sum(t, axis=-1) * inv_hw

    h = jax.lax.dot_general(
        pooled, w1_ref[...], (((1,), (1,)), ((), ())),
        preferred_element_type=jnp.float32) + b1_ref[...][None, :]
    a = alpha_ref[0]
    h = jnp.maximum(h, 0.0) + a * jnp.minimum(h, 0.0)
    z = jax.lax.dot_general(
        h, w2_ref[...], (((1,), (1,)), ((), ())),
        preferred_element_type=jnp.float32) + b2_ref[...][None, :]
    s = jax.nn.sigmoid(z).astype(o_ref.dtype)

    s4 = jax.lax.broadcast_in_dim(s, x.shape, (0, 1))
    o_ref[...] = x * s4


def kernel(x, w1, b1, w2, b2, alpha):
    B, C, H, W = x.shape
    hidden = w1.shape[0]
    dtype = x.dtype
    itemsize = jnp.dtype(dtype).itemsize

    W_pad = -(-W // _LANE) * _LANE
    H_pad = -(-H // _SUBLANE) * _SUBLANE
    row_bytes = C * H_pad * W_pad * itemsize

    bt = 1
    for cand in range(1, B + 1):
        if B % cand == 0 and cand * row_bytes <= _TARGET_BLOCK_BYTES:
            bt = cand
    nb = B // bt

    inv_hw = 1.0 / float(H * W)

    block_bytes = bt * row_bytes
    vmem = int(min(100 * 2**20, 4 * block_bytes + 6 * 2**20))

    import functools
    return pl.pallas_call(
        functools.partial(_soca4d_kernel, inv_hw),
        out_shape=jax.ShapeDtypeStruct((B, C, H, W), dtype),
        grid=(nb,),
        in_specs=[
            pl.BlockSpec(memory_space=pltpu.MemorySpace.SMEM),
            pl.BlockSpec((bt, C, H, W), lambda b: (b, 0, 0, 0)),
            pl.BlockSpec((hidden, C), lambda b: (0, 0)),
            pl.BlockSpec((hidden,), lambda b: (0,)),
            pl.BlockSpec((C, hidden), lambda b: (0, 0)),
            pl.BlockSpec((C,), lambda b: (0,)),
        ],
        out_specs=pl.BlockSpec((bt, C, H, W), lambda b: (b, 0, 0, 0)),
        compiler_params=pltpu.CompilerParams(
            dimension_semantics=("parallel",),
            vmem_limit_bytes=vmem),
    )(alpha, x, w1, b1, w2, b2)


# speedup vs baseline: 1.0129x; 1.0081x over previous
import functools

import jax
import jax.numpy as jnp
from jax.experimental import pallas as pl
from jax.experimental.pallas import tpu as pltpu

_LANE = 128
_SUBLANE = 8
_TARGET_BLOCK_BYTES = 8 * 2**20


def _soca4d_kernel(inv_hw, hidden, x_ref, p_ref, o_ref):
    x = x_ref[...]

    t = jnp.sum(x.astype(jnp.float32), axis=2)
    pooled = jnp.sum(t, axis=-1) * inv_hw

    p = p_ref[...]
    w1 = p[0:hidden, :]
    w2t = p[hidden:2 * hidden, :]
    b1 = p[2 * hidden:2 * hidden + 1, 0:hidden]
    b2 = p[2 * hidden + 1:2 * hidden + 2, :]
    a = p[2 * hidden + 2:2 * hidden + 3, 0:1]

    h = jax.lax.dot_general(
        pooled, w1, (((1,), (1,)), ((), ())),
        preferred_element_type=jnp.float32) + b1
    h = jnp.maximum(h, 0.0) + a * jnp.minimum(h, 0.0)
    z = jax.lax.dot_general(
        h, w2t, (((1,), (0,)), ((), ())),
        preferred_element_type=jnp.float32) + b2
    s = jax.nn.sigmoid(z).astype(o_ref.dtype)

    s4 = jax.lax.broadcast_in_dim(s, x.shape, (0, 1))
    o_ref[...] = x * s4


def kernel(x, w1, b1, w2, b2, alpha):
    B, C, H, W = x.shape
    hidden = w1.shape[0]
    dtype = x.dtype
    itemsize = jnp.dtype(dtype).itemsize

    W_pad = -(-W // _LANE) * _LANE
    H_pad = -(-H // _SUBLANE) * _SUBLANE
    row_bytes = C * H_pad * W_pad * itemsize

    bt = 1
    for cand in range(1, B + 1):
        if B % cand == 0 and cand * row_bytes <= _TARGET_BLOCK_BYTES:
            bt = cand
    nb = B // bt

    inv_hw = 1.0 / float(H * W)
    f32 = jnp.float32
    packed = jnp.concatenate([
        w1.astype(f32),
        w2.T.astype(f32),
        jnp.pad(b1.astype(f32), (0, C - hidden)).reshape(1, C),
        b2.astype(f32).reshape(1, C),
        jnp.pad(alpha.astype(f32), (0, C - 1)).reshape(1, C),
    ], axis=0)

    block_bytes = bt * row_bytes
    vmem = int(min(100 * 2**20, 4 * block_bytes + 6 * 2**20))

    return pl.pallas_call(
        functools.partial(_soca4d_kernel, inv_hw, hidden),
        out_shape=jax.ShapeDtypeStruct((B, C, H, W), dtype),
        grid=(nb,),
        in_specs=[
            pl.BlockSpec((bt, C, H, W), lambda b: (b, 0, 0, 0)),
            pl.BlockSpec((2 * hidden + 3, C), lambda b: (0, 0)),
        ],
        out_specs=pl.BlockSpec((bt, C, H, W), lambda b: (b, 0, 0, 0)),
        compiler_params=pltpu.CompilerParams(
            dimension_semantics=("parallel",),
            vmem_limit_bytes=vmem),
    )(x, packed)
